# Initial kernel scaffold; baseline (speedup 1.0000x reference)
import jax
import jax.numpy as jnp
from jax.experimental import pallas as pl


def kernel(x, edge_index, W1, b1, W2, b2, Wl1, bl1, Wl2, bl2):
    raise NotImplementedError("write your pallas kernel here")


# speedup vs baseline: 11.0873x; 11.0873x over previous
import functools

import jax
import jax.numpy as jnp
from jax import lax
from jax.experimental import pallas as pl
from jax.experimental.pallas import tpu as pltpu
from jax.experimental.pallas import tpu_sc as plsc

N_NODES = 10000
D = 256
HALF = 128
N_EDGES = 160000
EB = 128
N_BATCHES = N_EDGES // EB
R = 1000
NR = N_NODES // R

ROWS_BIG = 640
ROWS_SMALL = 624

_mesh = plsc.VectorSubcoreMesh(core_axis_name="c", subcore_axis_name="s")


def _row_range(s):
    start = jnp.where(s == 0, 0, ROWS_BIG + (s - 1) * ROWS_SMALL)
    nz16 = jnp.where(s == 0, ROWS_BIG // 16, ROWS_SMALL // 16)
    return start, nz16


@functools.partial(
    pl.kernel,
    out_type=jax.ShapeDtypeStruct((2, N_NODES, 16), jnp.float32),
    mesh=_mesh,
    scratch_types=[
        pltpu.VMEM((EB,), jnp.int32),
        pltpu.VMEM((EB, 16), jnp.float32),
        pltpu.VMEM((16, 16), jnp.float32),
        pltpu.VMEM_SHARED((N_NODES, 16), jnp.float32),
    ],
)
def _sc_degree(edge_hbm, out_hbm, idx_v, ones_v, zbuf_v, acc_sh):
    c = lax.axis_index("c")
    s = lax.axis_index("s")
    wid = s * 2 + c
    start, nz16 = _row_range(s)

    one = jnp.ones((16,), jnp.float32)
    zero = jnp.zeros((16,), jnp.float32)

    def fill_ones(i, _):
        ones_v[i, :] = one
        return 0

    lax.fori_loop(0, EB, fill_ones, 0)

    def fill_zero(i, _):
        zbuf_v[i, :] = zero
        return 0

    lax.fori_loop(0, 16, fill_zero, 0)

    def zero_acc(j, _):
        pltpu.sync_copy(zbuf_v, acc_sh.at[pl.ds(start + j * 16, 16)])
        return 0

    lax.fori_loop(0, nz16, zero_acc, 0)
    plsc.subcore_barrier()

    def body(t, _):
        k = t * 32 + wid

        @pl.when(k < N_BATCHES)
        def _():
            pltpu.sync_copy(edge_hbm.at[pl.ds(N_EDGES + k * EB, EB)], idx_v)
            pltpu.sync_copy(ones_v, acc_sh.at[idx_v], add=True)

        return 0

    lax.fori_loop(0, (N_BATCHES + 31) // 32, body, 0)
    plsc.subcore_barrier()

    @pl.when(s == 0)
    def _():
        pltpu.sync_copy(acc_sh.at[pl.ds(0, ROWS_BIG)],
                        out_hbm.at[c, pl.ds(0, ROWS_BIG)])

    @pl.when(s > 0)
    def _():
        pltpu.sync_copy(acc_sh.at[pl.ds(start, ROWS_SMALL)],
                        out_hbm.at[c, pl.ds(start, ROWS_SMALL)])


@functools.partial(
    pl.kernel,
    out_type=jax.ShapeDtypeStruct((2 * N_NODES, HALF), jnp.float32),
    mesh=_mesh,
    scratch_types=[
        pltpu.VMEM((EB,), jnp.int32),
        pltpu.VMEM((EB,), jnp.int32),
        pltpu.VMEM((EB,), jnp.int32),
        pltpu.VMEM((EB, HALF), jnp.float32),
        pltpu.VMEM((16, HALF), jnp.float32),
        pltpu.VMEM_SHARED((N_NODES, HALF), jnp.float32),
        pltpu.SemaphoreType.DMA,
    ],
)
def _sc_aggregate(g_hbm, edge_hbm, out_hbm,
                  src_v, srcoff_v, dst_v, rows_v, zbuf_v, acc_sh, sem):
    c = lax.axis_index("c")
    s = lax.axis_index("s")
    row_off = c * N_NODES
    start, nz16 = _row_range(s)

    zero = jnp.zeros((16,), jnp.float32)

    def fill_zero(i, _):
        for j in range(HALF // 16):
            zbuf_v[i, pl.ds(j * 16, 16)] = zero
        return 0

    lax.fori_loop(0, 16, fill_zero, 0)

    def zero_acc(j, _):
        pltpu.sync_copy(zbuf_v, acc_sh.at[pl.ds(start + j * 16, 16)])
        return 0

    lax.fori_loop(0, nz16, zero_acc, 0)
    plsc.subcore_barrier()

    def body(t, _):
        k = t * 16 + s

        @pl.when(k < N_BATCHES)
        def _():
            base = k * EB
            pltpu.sync_copy(edge_hbm.at[pl.ds(base, EB)], src_v)
            for j in range(EB // 16):
                srcoff_v[pl.ds(j * 16, 16)] = src_v[pl.ds(j * 16, 16)] + row_off
            gather = pltpu.async_copy(g_hbm.at[srcoff_v], rows_v, sem)
            pltpu.sync_copy(edge_hbm.at[pl.ds(N_EDGES + base, EB)], dst_v)
            gather.wait()
            pltpu.sync_copy(rows_v, acc_sh.at[dst_v], add=True)

        return 0

    lax.fori_loop(0, (N_BATCHES + 15) // 16, body, 0)
    plsc.subcore_barrier()

    @pl.when(s == 0)
    def _():
        pltpu.sync_copy(acc_sh.at[pl.ds(0, ROWS_BIG)],
                        out_hbm.at[pl.ds(row_off, ROWS_BIG)])

    @pl.when(s > 0)
    def _():
        pltpu.sync_copy(acc_sh.at[pl.ds(start, ROWS_SMALL)],
                        out_hbm.at[pl.ds(row_off + start, ROWS_SMALL)])


def _dinv_of(deg2_blk):
    deg = deg2_blk[0, :, 0] + deg2_blk[1, :, 0] + 1.0
    return lax.rsqrt(deg)[:, None]


def _tc_encode1_body(deg2_ref, x_ref, w_ref, g_ref):
    dinv = _dinv_of(deg2_ref[...])
    h = jnp.dot(x_ref[...], w_ref[...], preferred_element_type=jnp.float32)
    g_ref[...] = h * dinv


def _tc_encode1(deg2, x, W1):
    return pl.pallas_call(
        _tc_encode1_body,
        grid=(2, NR),
        in_specs=[
            pl.BlockSpec((2, R, 16), lambda c, r: (0, r, 0)),
            pl.BlockSpec((R, D), lambda c, r: (r, 0)),
            pl.BlockSpec((D, HALF), lambda c, r: (0, c)),
        ],
        out_specs=pl.BlockSpec((R, HALF), lambda c, r: (c * NR + r, 0)),
        out_shape=jax.ShapeDtypeStruct((2 * N_NODES, HALF), jnp.float32),
    )(deg2, x, W1)


def _tc_encode2_body(deg2_ref, slo_ref, shi_ref, glo_ref, ghi_ref,
                     b1_ref, w2_ref, g2_ref):
    dinv = _dinv_of(deg2_ref[...])
    r_lo = jnp.maximum(dinv * (slo_ref[...] + glo_ref[...]) + b1_ref[0, :HALF], 0.0)
    r_hi = jnp.maximum(dinv * (shi_ref[...] + ghi_ref[...]) + b1_ref[0, HALF:], 0.0)
    h = (jnp.dot(r_lo, w2_ref[:HALF, :], preferred_element_type=jnp.float32)
         + jnp.dot(r_hi, w2_ref[HALF:, :], preferred_element_type=jnp.float32))
    g2_ref[...] = h * dinv


def _tc_encode2(deg2, s1, g1, b1, W2):
    half_lo = pl.BlockSpec((R, HALF), lambda c, r: (r, 0))
    half_hi = pl.BlockSpec((R, HALF), lambda c, r: (NR + r, 0))
    return pl.pallas_call(
        _tc_encode2_body,
        grid=(2, NR),
        in_specs=[
            pl.BlockSpec((2, R, 16), lambda c, r: (0, r, 0)),
            half_lo, half_hi, half_lo, half_hi,
            pl.BlockSpec((1, D), lambda c, r: (0, 0)),
            pl.BlockSpec((D, HALF), lambda c, r: (0, c)),
        ],
        out_specs=pl.BlockSpec((R, HALF), lambda c, r: (c * NR + r, 0)),
        out_shape=jax.ShapeDtypeStruct((2 * N_NODES, HALF), jnp.float32),
    )(deg2, s1, s1, g1, g1, b1, W2)


def _tc_decode_body(deg2_ref, slo_ref, shi_ref, glo_ref, ghi_ref, b2_ref,
                    wl1_ref, bl1_ref, wl2_ref, bl2_ref, out_ref):
    dinv = _dinv_of(deg2_ref[...])
    h_lo = dinv * (slo_ref[...] + glo_ref[...]) + b2_ref[0, :HALF]
    h_hi = dinv * (shi_ref[...] + ghi_ref[...]) + b2_ref[0, HALF:]
    h = jnp.concatenate([h_lo, h_hi], axis=1)
    d = jnp.maximum(
        jnp.dot(h, wl1_ref[...], preferred_element_type=jnp.float32)
        + bl1_ref[0, :], 0.0)
    out_ref[...] = (jnp.dot(d, wl2_ref[...], preferred_element_type=jnp.float32)
                    + bl2_ref[0, :])


def _tc_decode(deg2, s2, g2, b2, Wl1, bl1, Wl2, bl2):
    half_lo = pl.BlockSpec((R, HALF), lambda r: (r, 0))
    half_hi = pl.BlockSpec((R, HALF), lambda r: (NR + r, 0))
    vec = pl.BlockSpec((1, D), lambda r: (0, 0))
    mat = pl.BlockSpec((D, D), lambda r: (0, 0))
    return pl.pallas_call(
        _tc_decode_body,
        grid=(NR,),
        in_specs=[
            pl.BlockSpec((2, R, 16), lambda r: (0, r, 0)),
            half_lo, half_hi, half_lo, half_hi,
            vec, mat, vec, mat, vec,
        ],
        out_specs=pl.BlockSpec((R, D), lambda r: (r, 0)),
        out_shape=jax.ShapeDtypeStruct((N_NODES, D), jnp.float32),
    )(deg2, s2, s2, g2, g2, b2, Wl1, bl1, Wl2, bl2)


def kernel(x, edge_index, W1, b1, W2, b2, Wl1, bl1, Wl2, bl2):
    edge_flat = edge_index.astype(jnp.int32).reshape(-1)
    b1r = b1.reshape(1, D)
    b2r = b2.reshape(1, D)
    bl1r = bl1.reshape(1, D)
    bl2r = bl2.reshape(1, D)

    deg2 = _sc_degree(edge_flat)
    g1 = _tc_encode1(deg2, x, W1)
    s1 = _sc_aggregate(g1, edge_flat)
    g2 = _tc_encode2(deg2, s1, g1, b1r, W2)
    s2 = _sc_aggregate(g2, edge_flat)
    return _tc_decode(deg2, s2, g2, b2r, Wl1, bl1r, Wl2, bl2r)


# speedup vs baseline: 12.8384x; 1.1579x over previous
import functools

import jax
import jax.numpy as jnp
from jax import lax
from jax.experimental import pallas as pl
from jax.experimental.pallas import tpu as pltpu
from jax.experimental.pallas import tpu_sc as plsc

N_NODES = 10000
D = 256
HALF = 128
N_EDGES = 160000
EB = 128
N_BATCHES = N_EDGES // EB
R = 1000
NR = N_NODES // R

NB_MAX = 79
NBD_MAX = 40

ROWS_BIG = 640
ROWS_SMALL = 624

_mesh = plsc.VectorSubcoreMesh(core_axis_name="c", subcore_axis_name="s")


def _row_range(s):
    start = jnp.where(s == 0, 0, ROWS_BIG + (s - 1) * ROWS_SMALL)
    nz16 = jnp.where(s == 0, ROWS_BIG // 16, ROWS_SMALL // 16)
    return start, nz16


@functools.partial(
    pl.kernel,
    out_type=jax.ShapeDtypeStruct((2, N_NODES, HALF), jnp.float32),
    mesh=_mesh,
    scratch_types=[
        pltpu.VMEM((EB,), jnp.int32),
        pltpu.VMEM((EB, HALF), jnp.float32),
        pltpu.VMEM((16, HALF), jnp.float32),
        pltpu.VMEM_SHARED((N_NODES, HALF), jnp.float32),
    ],
)
def _sc_degree(edge_hbm, ones_hbm, out_hbm, idx_v, ones_v, zbuf_v, acc_sh):
    c = lax.axis_index("c")
    s = lax.axis_index("s")
    wid = s * 2 + c
    start, nz16 = _row_range(s)

    pltpu.sync_copy(ones_hbm.at[pl.ds(0, EB)], ones_v)
    pltpu.sync_copy(ones_hbm.at[pl.ds(EB, 16)], zbuf_v)

    def zero_acc(j, _):
        pltpu.sync_copy(zbuf_v, acc_sh.at[pl.ds(start + j * 16, 16)])
        return 0

    lax.fori_loop(0, nz16, zero_acc, 0)
    plsc.subcore_barrier()

    def body(t, _):
        k = t * 32 + wid

        @pl.when(k < N_BATCHES)
        def _():
            pltpu.sync_copy(edge_hbm.at[pl.ds(N_EDGES + k * EB, EB)], idx_v)
            pltpu.sync_copy(ones_v, acc_sh.at[idx_v], add=True)

        return 0

    lax.fori_loop(0, (N_BATCHES + 31) // 32, body, 0)
    plsc.subcore_barrier()

    @pl.when(s == 0)
    def _():
        pltpu.sync_copy(acc_sh.at[pl.ds(0, ROWS_BIG)],
                        out_hbm.at[c, pl.ds(0, ROWS_BIG)])

    @pl.when(s > 0)
    def _():
        pltpu.sync_copy(acc_sh.at[pl.ds(start, ROWS_SMALL)],
                        out_hbm.at[c, pl.ds(start, ROWS_SMALL)])


@functools.partial(
    pl.kernel,
    out_type=jax.ShapeDtypeStruct((2 * N_NODES, HALF), jnp.float32),
    mesh=_mesh,
    scratch_types=[
        pltpu.VMEM((NB_MAX * EB,), jnp.int32),
        pltpu.VMEM((EB,), jnp.int32),
        pltpu.VMEM((EB,), jnp.int32),
        pltpu.VMEM((EB,), jnp.int32),
        pltpu.VMEM((EB,), jnp.int32),
        pltpu.VMEM((EB, HALF), jnp.float32),
        pltpu.VMEM((EB, HALF), jnp.float32),
        pltpu.VMEM((8, HALF), jnp.float32),
        pltpu.VMEM_SHARED((N_NODES, HALF), jnp.float32),
        pltpu.SemaphoreType.DMA,
        pltpu.SemaphoreType.DMA,
        pltpu.SemaphoreType.DMA,
        pltpu.SemaphoreType.DMA,
    ],
)
def _sc_aggregate(g_hbm, edge_hbm, out_hbm,
                  srcbuf_v, srcb0_v, srcb1_v, dstb0_v, dstb1_v,
                  rows0_v, rows1_v, zbuf_v, acc_sh,
                  sem0, sem1, semd0, semd1):
    c = lax.axis_index("c")
    s = lax.axis_index("s")
    row_off = c * N_NODES
    k0 = s * 78 + jnp.minimum(s, 2)
    nb = 78 + (s < 2).astype(jnp.int32)
    start, nz16 = _row_range(s)

    @pl.when(s < 2)
    def _():
        pltpu.sync_copy(edge_hbm.at[pl.ds(k0 * EB, 79 * EB)],
                        srcbuf_v.at[pl.ds(0, 79 * EB)])

    @pl.when(s >= 2)
    def _():
        pltpu.sync_copy(edge_hbm.at[pl.ds(k0 * EB, 78 * EB)],
                        srcbuf_v.at[pl.ds(0, 78 * EB)])

    def addoff(j, _):
        srcbuf_v[pl.ds(j * 16, 16)] = srcbuf_v[pl.ds(j * 16, 16)] + row_off
        return 0

    lax.fori_loop(0, NB_MAX * EB // 16, addoff, 0)

    zero = jnp.zeros((16,), jnp.float32)

    def fill_zero(i, _):
        for j in range(HALF // 16):
            zbuf_v[i, pl.ds(j * 16, 16)] = zero
        return 0

    lax.fori_loop(0, 8, fill_zero, 0)

    def zero_acc(j, _):
        pltpu.sync_copy(zbuf_v, acc_sh.at[pl.ds(start + j * 8, 8)])
        return 0

    lax.fori_loop(0, nz16 * 2, zero_acc, 0)
    plsc.subcore_barrier()

    G = 16
    srcb = (srcb0_v, srcb1_v)
    dstb = (dstb0_v, dstb1_v)
    rows = (rows0_v, rows1_v)
    sems = (sem0, sem1)
    semd = (semd0, semd1)

    def issue(t, p):
        for j in range(EB // 16):
            srcb[p][pl.ds(j * 16, 16)] = srcbuf_v[pl.ds(t * EB + j * 16, 16)]
        g = pltpu.async_copy(g_hbm.at[srcb[p]], rows[p], sems[p])
        d = pltpu.async_copy(
            edge_hbm.at[pl.ds(N_EDGES + (k0 + t) * EB, EB)], dstb[p], semd[p])
        return g, d

    def pairbody(u, _):
        t0 = 2 * u
        t1 = t0 + 1
        g0, d0 = issue(t0, 0)
        g1, d1 = issue(t1, 1)
        g0.wait()
        d0.wait()
        g1.wait()
        d1.wait()
        pltpu.sync_copy(rows0_v, acc_sh.at[dstb0_v], add=True)
        pltpu.sync_copy(rows1_v, acc_sh.at[dstb1_v], add=True)
        return 0

    lax.fori_loop(0, 39, pairbody, 0)

    @pl.when(nb > 78)
    def _():
        g, d = issue(78, 0)
        g.wait()
        d.wait()
        pltpu.sync_copy(rows0_v, acc_sh.at[dstb0_v], add=True)
    plsc.subcore_barrier()

    @pl.when(s == 0)
    def _():
        pltpu.sync_copy(acc_sh.at[pl.ds(0, ROWS_BIG)],
                        out_hbm.at[pl.ds(row_off, ROWS_BIG)])

    @pl.when(s > 0)
    def _():
        pltpu.sync_copy(acc_sh.at[pl.ds(start, ROWS_SMALL)],
                        out_hbm.at[pl.ds(row_off + start, ROWS_SMALL)])


def _dinv_of(deg2_blk):
    deg = deg2_blk[0, :, 0] + deg2_blk[1, :, 0] + 1.0
    return lax.rsqrt(deg)[:, None]


def _tc_encode1_body(deg2_ref, x_ref, w_ref, g_ref):
    dinv = _dinv_of(deg2_ref[...])
    h = jnp.dot(x_ref[...], w_ref[...], preferred_element_type=jnp.float32)
    g_ref[...] = h * dinv


def _tc_encode1(deg2, x, W1):
    return pl.pallas_call(
        _tc_encode1_body,
        grid=(2, NR),
        in_specs=[
            pl.BlockSpec((2, R, HALF), lambda c, r: (0, r, 0)),
            pl.BlockSpec((R, D), lambda c, r: (r, 0)),
            pl.BlockSpec((D, HALF), lambda c, r: (0, c)),
        ],
        out_specs=pl.BlockSpec((R, HALF), lambda c, r: (c * NR + r, 0)),
        out_shape=jax.ShapeDtypeStruct((2 * N_NODES, HALF), jnp.float32),
    )(deg2, x, W1)


def _tc_encode2_body(deg2_ref, slo_ref, shi_ref, glo_ref, ghi_ref,
                     b1_ref, w2_ref, g2_ref):
    dinv = _dinv_of(deg2_ref[...])
    r_lo = jnp.maximum(dinv * (slo_ref[...] + glo_ref[...]) + b1_ref[0, :HALF], 0.0)
    r_hi = jnp.maximum(dinv * (shi_ref[...] + ghi_ref[...]) + b1_ref[0, HALF:], 0.0)
    h = (jnp.dot(r_lo, w2_ref[:HALF, :], preferred_element_type=jnp.float32)
         + jnp.dot(r_hi, w2_ref[HALF:, :], preferred_element_type=jnp.float32))
    g2_ref[...] = h * dinv


def _tc_encode2(deg2, s1, g1, b1, W2):
    half_lo = pl.BlockSpec((R, HALF), lambda c, r: (r, 0))
    half_hi = pl.BlockSpec((R, HALF), lambda c, r: (NR + r, 0))
    return pl.pallas_call(
        _tc_encode2_body,
        grid=(2, NR),
        in_specs=[
            pl.BlockSpec((2, R, HALF), lambda c, r: (0, r, 0)),
            half_lo, half_hi, half_lo, half_hi,
            pl.BlockSpec((1, D), lambda c, r: (0, 0)),
            pl.BlockSpec((D, HALF), lambda c, r: (0, c)),
        ],
        out_specs=pl.BlockSpec((R, HALF), lambda c, r: (c * NR + r, 0)),
        out_shape=jax.ShapeDtypeStruct((2 * N_NODES, HALF), jnp.float32),
    )(deg2, s1, s1, g1, g1, b1, W2)


def _tc_decode_body(deg2_ref, slo_ref, shi_ref, glo_ref, ghi_ref, b2_ref,
                    wl1_ref, bl1_ref, wl2_ref, bl2_ref, out_ref):
    dinv = _dinv_of(deg2_ref[...])
    h_lo = dinv * (slo_ref[...] + glo_ref[...]) + b2_ref[0, :HALF]
    h_hi = dinv * (shi_ref[...] + ghi_ref[...]) + b2_ref[0, HALF:]
    h = jnp.concatenate([h_lo, h_hi], axis=1)
    d = jnp.maximum(
        jnp.dot(h, wl1_ref[...], preferred_element_type=jnp.float32)
        + bl1_ref[0, :], 0.0)
    out_ref[...] = (jnp.dot(d, wl2_ref[...], preferred_element_type=jnp.float32)
                    + bl2_ref[0, :])


def _tc_decode(deg2, s2, g2, b2, Wl1, bl1, Wl2, bl2):
    half_lo = pl.BlockSpec((R, HALF), lambda r: (r, 0))
    half_hi = pl.BlockSpec((R, HALF), lambda r: (NR + r, 0))
    vec = pl.BlockSpec((1, D), lambda r: (0, 0))
    mat = pl.BlockSpec((D, D), lambda r: (0, 0))
    return pl.pallas_call(
        _tc_decode_body,
        grid=(NR,),
        in_specs=[
            pl.BlockSpec((2, R, HALF), lambda r: (0, r, 0)),
            half_lo, half_hi, half_lo, half_hi,
            vec, mat, vec, mat, vec,
        ],
        out_specs=pl.BlockSpec((R, D), lambda r: (r, 0)),
        out_shape=jax.ShapeDtypeStruct((N_NODES, D), jnp.float32),
    )(deg2, s2, s2, g2, g2, b2, Wl1, bl1, Wl2, bl2)


def kernel(x, edge_index, W1, b1, W2, b2, Wl1, bl1, Wl2, bl2):
    edge_flat = edge_index.astype(jnp.int32).reshape(-1)
    b1r = b1.reshape(1, D)
    b2r = b2.reshape(1, D)
    bl1r = bl1.reshape(1, D)
    bl2r = bl2.reshape(1, D)

    ones_c = jnp.concatenate([jnp.ones((EB, HALF), jnp.float32),
                              jnp.zeros((16, HALF), jnp.float32)])
    deg2 = _sc_degree(edge_flat, ones_c)
    g1 = _tc_encode1(deg2, x, W1)
    s1 = _sc_aggregate(g1, edge_flat)
    g2 = _tc_encode2(deg2, s1, g1, b1r, W2)
    s2 = _sc_aggregate(g2, edge_flat)
    return _tc_decode(deg2, s2, g2, b2r, Wl1, bl1r, Wl2, bl2r)


# speedup vs baseline: 13.0258x; 1.0146x over previous
import functools

import jax
import jax.numpy as jnp
from jax import lax
from jax.experimental import pallas as pl
from jax.experimental.pallas import tpu as pltpu
from jax.experimental.pallas import tpu_sc as plsc

N_NODES = 10000
D = 256
HALF = 128
N_EDGES = 160000
EB = 128
N_BATCHES = N_EDGES // EB
R = 1000
NR = N_NODES // R

NB_MAX = 79
NBD_MAX = 40

ROWS_BIG = 640
ROWS_SMALL = 624

DEG_W = 128

_mesh = plsc.VectorSubcoreMesh(core_axis_name="c", subcore_axis_name="s")


def _row_range(s):
    start = jnp.where(s == 0, 0, ROWS_BIG + (s - 1) * ROWS_SMALL)
    nz16 = jnp.where(s == 0, ROWS_BIG // 16, ROWS_SMALL // 16)
    return start, nz16


@functools.partial(
    pl.kernel,
    out_type=jax.ShapeDtypeStruct((2, N_NODES, DEG_W), jnp.float32),
    mesh=_mesh,
    scratch_types=[
        pltpu.VMEM((EB,), jnp.int32),
        pltpu.VMEM((EB, DEG_W), jnp.float32),
        pltpu.VMEM((16, DEG_W), jnp.float32),
        pltpu.VMEM_SHARED((N_NODES, DEG_W), jnp.float32),
    ],
)
def _sc_degree(edge_hbm, ones_hbm, out_hbm, idx_v, ones_v, zbuf_v, acc_sh):
    c = lax.axis_index("c")
    s = lax.axis_index("s")
    wid = s * 2 + c
    start, nz16 = _row_range(s)

    pltpu.sync_copy(ones_hbm.at[pl.ds(0, EB)], ones_v)
    pltpu.sync_copy(ones_hbm.at[pl.ds(EB, 16)], zbuf_v)

    def zero_acc(j, _):
        pltpu.sync_copy(zbuf_v, acc_sh.at[pl.ds(start + j * 16, 16)])
        return 0

    lax.fori_loop(0, nz16, zero_acc, 0)
    plsc.subcore_barrier()

    def body(t, _):
        k = t * 32 + wid

        @pl.when(k < N_BATCHES)
        def _():
            pltpu.sync_copy(edge_hbm.at[pl.ds(N_EDGES + k * EB, EB)], idx_v)
            pltpu.sync_copy(ones_v, acc_sh.at[idx_v], add=True)

        return 0

    lax.fori_loop(0, (N_BATCHES + 31) // 32, body, 0)
    plsc.subcore_barrier()

    @pl.when(s == 0)
    def _():
        pltpu.sync_copy(acc_sh.at[pl.ds(0, ROWS_BIG)],
                        out_hbm.at[c, pl.ds(0, ROWS_BIG)])

    @pl.when(s > 0)
    def _():
        pltpu.sync_copy(acc_sh.at[pl.ds(start, ROWS_SMALL)],
                        out_hbm.at[c, pl.ds(start, ROWS_SMALL)])


@functools.partial(
    pl.kernel,
    out_type=jax.ShapeDtypeStruct((2 * N_NODES, HALF), jnp.float32),
    mesh=_mesh,
    scratch_types=[
        pltpu.VMEM((NB_MAX * EB,), jnp.int32),
        pltpu.VMEM((EB,), jnp.int32),
        pltpu.VMEM((EB,), jnp.int32),
        pltpu.VMEM((EB,), jnp.int32),
        pltpu.VMEM((EB,), jnp.int32),
        pltpu.VMEM((EB, HALF), jnp.float32),
        pltpu.VMEM((EB, HALF), jnp.float32),
        pltpu.VMEM((8, HALF), jnp.float32),
        pltpu.VMEM_SHARED((N_NODES, HALF), jnp.float32),
        pltpu.SemaphoreType.DMA,
        pltpu.SemaphoreType.DMA,
        pltpu.SemaphoreType.DMA,
        pltpu.SemaphoreType.DMA,
    ],
)
def _sc_aggregate(g_hbm, edge_hbm, out_hbm,
                  srcbuf_v, srcb0_v, srcb1_v, dstb0_v, dstb1_v,
                  rows0_v, rows1_v, zbuf_v, acc_sh,
                  sem0, sem1, semd0, semd1):
    c = lax.axis_index("c")
    s = lax.axis_index("s")
    row_off = c * N_NODES
    k0 = s * 78 + jnp.minimum(s, 2)
    nb = 78 + (s < 2).astype(jnp.int32)
    start, nz16 = _row_range(s)

    @pl.when(s < 2)
    def _():
        pltpu.sync_copy(edge_hbm.at[pl.ds(k0 * EB, 79 * EB)],
                        srcbuf_v.at[pl.ds(0, 79 * EB)])

    @pl.when(s >= 2)
    def _():
        pltpu.sync_copy(edge_hbm.at[pl.ds(k0 * EB, 78 * EB)],
                        srcbuf_v.at[pl.ds(0, 78 * EB)])

    def addoff(j, _):
        srcbuf_v[pl.ds(j * 16, 16)] = srcbuf_v[pl.ds(j * 16, 16)] + row_off
        return 0

    lax.fori_loop(0, NB_MAX * EB // 16, addoff, 0)

    zero = jnp.zeros((16,), jnp.float32)

    def fill_zero(i, _):
        for j in range(HALF // 16):
            zbuf_v[i, pl.ds(j * 16, 16)] = zero
        return 0

    lax.fori_loop(0, 8, fill_zero, 0)

    def zero_acc(j, _):
        pltpu.sync_copy(zbuf_v, acc_sh.at[pl.ds(start + j * 8, 8)])
        return 0

    lax.fori_loop(0, nz16 * 2, zero_acc, 0)
    plsc.subcore_barrier()

    G = 16
    srcb = (srcb0_v, srcb1_v)
    dstb = (dstb0_v, dstb1_v)
    rows = (rows0_v, rows1_v)
    sems = (sem0, sem1)
    semd = (semd0, semd1)

    def issue(t, p):
        for j in range(EB // 16):
            srcb[p][pl.ds(j * 16, 16)] = srcbuf_v[pl.ds(t * EB + j * 16, 16)]
        g = pltpu.async_copy(g_hbm.at[srcb[p]], rows[p], sems[p])
        d = pltpu.async_copy(
            edge_hbm.at[pl.ds(N_EDGES + (k0 + t) * EB, EB)], dstb[p], semd[p])
        return g, d

    def pairbody(u, _):
        t0 = 2 * u
        t1 = t0 + 1
        g0, d0 = issue(t0, 0)
        g1, d1 = issue(t1, 1)
        g0.wait()
        d0.wait()
        pltpu.sync_copy(rows0_v, acc_sh.at[dstb0_v], add=True)
        g1.wait()
        d1.wait()
        pltpu.sync_copy(rows1_v, acc_sh.at[dstb1_v], add=True)
        return 0

    lax.fori_loop(0, 39, pairbody, 0)

    @pl.when(nb > 78)
    def _():
        g, d = issue(78, 0)
        g.wait()
        d.wait()
        pltpu.sync_copy(rows0_v, acc_sh.at[dstb0_v], add=True)
    plsc.subcore_barrier()

    @pl.when(s == 0)
    def _():
        pltpu.sync_copy(acc_sh.at[pl.ds(0, ROWS_BIG)],
                        out_hbm.at[pl.ds(row_off, ROWS_BIG)])

    @pl.when(s > 0)
    def _():
        pltpu.sync_copy(acc_sh.at[pl.ds(start, ROWS_SMALL)],
                        out_hbm.at[pl.ds(row_off + start, ROWS_SMALL)])


def _dinv_of(deg2_blk):
    deg = deg2_blk[0, :, 0] + deg2_blk[1, :, 0] + 1.0
    return lax.rsqrt(deg)[:, None]


def _tc_encode1_body(deg2_ref, x_ref, w_ref, g_ref):
    dinv = _dinv_of(deg2_ref[...])
    h = jnp.dot(x_ref[...], w_ref[...], preferred_element_type=jnp.float32)
    g_ref[...] = h * dinv


def _tc_encode1(deg2, x, W1):
    return pl.pallas_call(
        _tc_encode1_body,
        grid=(2, NR),
        in_specs=[
            pl.BlockSpec((2, R, DEG_W), lambda c, r: (0, r, 0)),
            pl.BlockSpec((R, D), lambda c, r: (r, 0)),
            pl.BlockSpec((D, HALF), lambda c, r: (0, c)),
        ],
        out_specs=pl.BlockSpec((R, HALF), lambda c, r: (c * NR + r, 0)),
        out_shape=jax.ShapeDtypeStruct((2 * N_NODES, HALF), jnp.float32),
    )(deg2, x, W1)


def _tc_encode2_body(deg2_ref, slo_ref, shi_ref, glo_ref, ghi_ref,
                     b1_ref, w2_ref, g2_ref):
    dinv = _dinv_of(deg2_ref[...])
    r_lo = jnp.maximum(dinv * (slo_ref[...] + glo_ref[...]) + b1_ref[0, :HALF], 0.0)
    r_hi = jnp.maximum(dinv * (shi_ref[...] + ghi_ref[...]) + b1_ref[0, HALF:], 0.0)
    h = (jnp.dot(r_lo, w2_ref[:HALF, :], preferred_element_type=jnp.float32)
         + jnp.dot(r_hi, w2_ref[HALF:, :], preferred_element_type=jnp.float32))
    g2_ref[...] = h * dinv


def _tc_encode2(deg2, s1, g1, b1, W2):
    half_lo = pl.BlockSpec((R, HALF), lambda c, r: (r, 0))
    half_hi = pl.BlockSpec((R, HALF), lambda c, r: (NR + r, 0))
    return pl.pallas_call(
        _tc_encode2_body,
        grid=(2, NR),
        in_specs=[
            pl.BlockSpec((2, R, DEG_W), lambda c, r: (0, r, 0)),
            half_lo, half_hi, half_lo, half_hi,
            pl.BlockSpec((1, D), lambda c, r: (0, 0)),
            pl.BlockSpec((D, HALF), lambda c, r: (0, c)),
        ],
        out_specs=pl.BlockSpec((R, HALF), lambda c, r: (c * NR + r, 0)),
        out_shape=jax.ShapeDtypeStruct((2 * N_NODES, HALF), jnp.float32),
    )(deg2, s1, s1, g1, g1, b1, W2)


def _tc_decode_body(deg2_ref, slo_ref, shi_ref, glo_ref, ghi_ref, b2_ref,
                    wl1_ref, bl1_ref, wl2_ref, bl2_ref, out_ref):
    dinv = _dinv_of(deg2_ref[...])
    h_lo = dinv * (slo_ref[...] + glo_ref[...]) + b2_ref[0, :HALF]
    h_hi = dinv * (shi_ref[...] + ghi_ref[...]) + b2_ref[0, HALF:]
    h = jnp.concatenate([h_lo, h_hi], axis=1)
    d = jnp.maximum(
        jnp.dot(h, wl1_ref[...], preferred_element_type=jnp.float32)
        + bl1_ref[0, :], 0.0)
    out_ref[...] = (jnp.dot(d, wl2_ref[...], preferred_element_type=jnp.float32)
                    + bl2_ref[0, :])


def _tc_decode(deg2, s2, g2, b2, Wl1, bl1, Wl2, bl2):
    half_lo = pl.BlockSpec((R, HALF), lambda r: (r, 0))
    half_hi = pl.BlockSpec((R, HALF), lambda r: (NR + r, 0))
    vec = pl.BlockSpec((1, D), lambda r: (0, 0))
    mat = pl.BlockSpec((D, D), lambda r: (0, 0))
    return pl.pallas_call(
        _tc_decode_body,
        grid=(NR,),
        in_specs=[
            pl.BlockSpec((2, R, DEG_W), lambda r: (0, r, 0)),
            half_lo, half_hi, half_lo, half_hi,
            vec, mat, vec, mat, vec,
        ],
        out_specs=pl.BlockSpec((R, D), lambda r: (r, 0)),
        out_shape=jax.ShapeDtypeStruct((N_NODES, D), jnp.float32),
    )(deg2, s2, s2, g2, g2, b2, Wl1, bl1, Wl2, bl2)


def kernel(x, edge_index, W1, b1, W2, b2, Wl1, bl1, Wl2, bl2):
    edge_flat = edge_index.astype(jnp.int32).reshape(-1)
    b1r = b1.reshape(1, D)
    b2r = b2.reshape(1, D)
    bl1r = bl1.reshape(1, D)
    bl2r = bl2.reshape(1, D)

    ones_c = jnp.concatenate([jnp.ones((EB, DEG_W), jnp.float32),
                              jnp.zeros((16, DEG_W), jnp.float32)])
    deg2 = _sc_degree(edge_flat, ones_c)
    g1 = _tc_encode1(deg2, x, W1)
    s1 = _sc_aggregate(g1, edge_flat)
    g2 = _tc_encode2(deg2, s1, g1, b1r, W2)
    s2 = _sc_aggregate(g2, edge_flat)
    return _tc_decode(deg2, s2, g2, b2r, Wl1, bl1r, Wl2, bl2r)
